# baseline (speedup 1.0000x reference)
import numpy as np
import jax
import jax.numpy as jnp
from jax.experimental import pallas as pl

_NUM_CLASSES = 21
_THRESHOLD = 0.5
_NEGPOS_RATIO = 3
_V0 = 0.1
_V1 = 0.2
_B, _P, _C, _O = 32, 8732, 21, 16
_PP = 8736
_S, _L = 8, 1092
_M = 4
_SIGN = -(2 ** 31)


def _signed_key(i):
    return i ^ ((i >> 31) & 0x7FFFFFFF)


def _mbl_body(tgt_ref, prior_ref, loc_ref, conf_ref,
              out_l, out_r, out_c, out_n):
    g = pl.program_id(0)

    @pl.when(g == 0)
    def _init():
        z = jnp.zeros((1, 1), jnp.float32)
        out_l[...] = z
        out_r[...] = z
        out_c[...] = z
        out_n[...] = z

    f32 = jnp.float32
    i32 = jnp.int32
    pr = prior_ref[...]
    pcx = pr[0:1]
    pcy = pr[1:2]
    pw = pr[2:3]
    ph = pr[3:4]
    px1 = pcx - pw * 0.5
    py1 = pcy - ph * 0.5
    px2 = pcx + pw * 0.5
    py2 = pcy + ph * 0.5
    parea = (px2 - px1) * (py2 - py1)

    lin = (jax.lax.broadcasted_iota(i32, (1, _S, _L), 1) * _L
           + jax.lax.broadcasted_iota(i32, (1, _S, _L), 2))

    def tcol(t, col):
        return tgt_ref[:, t:t + 1, col:col + 1]

    neg1 = jnp.full((_M, _S, _L), -1.0, f32)
    v1 = neg1
    v2 = neg1
    i1 = jnp.zeros((_M, _S, _L), i32)
    i2 = jnp.zeros((_M, _S, _L), i32)
    t_match = jnp.full((_M, _S, _L), -1, i32)
    big = jnp.int32(_PP)
    for t in range(_O):
        tx1 = tcol(t, 0)
        ty1 = tcol(t, 1)
        tx2 = tcol(t, 2)
        ty2 = tcol(t, 3)
        tarea = (tx2 - tx1) * (ty2 - ty1)
        ix = jnp.clip(jnp.minimum(tx2, px2) - jnp.maximum(tx1, px1), 0.0, None)
        iy = jnp.clip(jnp.minimum(ty2, py2) - jnp.maximum(ty1, py1), 0.0, None)
        inter = ix * iy
        ov = inter / (tarea + parea - inter)
        upd1 = ov > v1
        upd2 = jnp.logical_and(ov > v2, jnp.logical_not(upd1))
        v2 = jnp.where(upd1, v1, jnp.where(upd2, ov, v2))
        i2 = jnp.where(upd1, i1, jnp.where(upd2, t, i2))
        v1 = jnp.where(upd1, ov, v1)
        i1 = jnp.where(upd1, t, i1)
        m_t = jnp.max(ov, axis=(1, 2), keepdims=True)
        bp_lin = jnp.min(jnp.where(ov == m_t, lin, big),
                         axis=(1, 2), keepdims=True)
        t_match = jnp.where(lin == bp_lin, t, t_match)

    forced = t_match >= 0
    ovl = jnp.where(forced, 2.0, v1)
    idxf = jnp.where(forced, t_match, i1)
    sb_idx = i2

    def gather_tree(idx, col):
        bit0 = (idx & 1) == 1
        bit1 = (idx & 2) == 2
        bit2 = (idx & 4) == 4
        bit3 = (idx & 8) == 8
        lvl = [jnp.where(bit0, tcol(2 * j + 1, col), tcol(2 * j, col))
               for j in range(8)]
        lvl = [jnp.where(bit1, lvl[2 * j + 1], lvl[2 * j]) for j in range(4)]
        lvl = [jnp.where(bit2, lvl[2 * j + 1], lvl[2 * j]) for j in range(2)]
        return jnp.where(bit3, lvl[1], lvl[0])

    mx1 = gather_tree(idxf, 0)
    my1 = gather_tree(idxf, 1)
    mx2 = gather_tree(idxf, 2)
    my2 = gather_tree(idxf, 3)
    lbl = gather_tree(idxf, 4)
    gx1 = gather_tree(sb_idx, 0)
    gy1 = gather_tree(sb_idx, 1)
    gx2 = gather_tree(sb_idx, 2)
    gy2 = gather_tree(sb_idx, 3)

    pos = ovl >= _THRESHOLD
    posf = pos.astype(f32)
    npos = jnp.sum(posf, axis=(1, 2), keepdims=True)

    ecx = ((mx1 + mx2) * 0.5 - pcx) / (_V0 * pw)
    ecy = ((my1 + my2) * 0.5 - pcy) / (_V0 * ph)
    ew = jnp.log((mx2 - mx1) / pw) / _V1
    eh = jnp.log((my2 - my1) / ph) / _V1
    ld0 = loc_ref[:, 0]
    ld1 = loc_ref[:, 1]
    ld2 = loc_ref[:, 2]
    ld3 = loc_ref[:, 3]

    sl1 = jnp.zeros((_M, _S, _L), f32)
    for ld, e in ((ld0, ecx), (ld1, ecy), (ld2, ew), (ld3, eh)):
        d = ld - e
        ad = jnp.abs(d)
        sl1 = sl1 + jnp.where(ad < 1.0, 0.5 * d * d, ad - 0.5)
    loss_l = jnp.sum(sl1 * posf)

    dcx = pcx + (ld0 * _V0) * pw
    dcy = pcy + (ld1 * _V0) * ph
    dw = pw * jnp.exp(ld2 * _V1)
    dh = ph * jnp.exp(ld3 * _V1)
    dx1 = dcx - dw * 0.5
    dx2 = dx1 + dw
    dy1 = dcy - dh * 0.5
    dy2 = dy1 + dh
    iw = jnp.clip(jnp.minimum(gx2, dx2) - jnp.maximum(gx1, dx1), 0.0, None)
    ih = jnp.clip(jnp.minimum(gy2, dy2) - jnp.maximum(gy1, dy1), 0.0, None)
    garea = (gx2 - gx1) * (gy2 - gy1)
    iog = (iw * ih) / garea
    rep = -jnp.log(jnp.maximum(1.0 - iog, 0.0) + jnp.float32(1e-10))
    loss_r = jnp.sum(rep * posf)

    cmax = conf_ref[:, 0]
    for c in range(1, _C):
        cmax = jnp.maximum(cmax, conf_ref[:, c])
    ssum = jnp.zeros((_M, _S, _L), f32)
    for c in range(_C):
        ssum = ssum + jnp.exp(conf_ref[:, c] - cmax)
    lse = jnp.log(ssum) + cmax
    conf_t = jnp.where(pos, lbl.astype(i32) + 1, 0)
    cb0 = (conf_t & 1) == 1
    cb1 = (conf_t & 2) == 2
    cb2 = (conf_t & 4) == 4
    cb3 = (conf_t & 8) == 8
    cb4 = (conf_t & 16) == 16
    lvl = [jnp.where(cb0, conf_ref[:, min(2 * j + 1, _C - 1)],
                     conf_ref[:, 2 * j]) for j in range(11)]
    lvl = [jnp.where(cb1, lvl[2 * j + 1], lvl[2 * j]) for j in range(5)] \
        + [lvl[10]]
    lvl = [jnp.where(cb2, lvl[2 * j + 1], lvl[2 * j]) for j in range(3)]
    lvl = [jnp.where(cb3, lvl[1], lvl[0]), lvl[2]]
    csel = jnp.where(cb4, lvl[1], lvl[0])
    ce = lse - csel
    zero = jnp.zeros((), f32)
    ce_pos = jnp.sum(jnp.where(pos, ce, zero))

    valid = lin < _P
    v = jnp.where(valid, jnp.where(pos, zero, ce), neg1)
    keys = _signed_key(jax.lax.bitcast_convert_type(v, i32))
    npos_i = npos.astype(i32)
    num_neg = jnp.minimum(_NEGPOS_RATIO * npos_i, _P - 1)

    x_bits = jnp.zeros((_M, 1, 1), i32)
    for bit in range(31, -1, -1):
        m = 1 << bit
        if m >= 2 ** 31:
            m -= 2 ** 32
        trial = x_bits | jnp.int32(m)
        trial_s = trial ^ jnp.int32(_SIGN)
        cnt = jnp.sum((keys >= trial_s).astype(i32),
                      axis=(1, 2), keepdims=True)
        x_bits = jnp.where(cnt >= num_neg, trial, x_bits)
    t_s = x_bits ^ jnp.int32(_SIGN)

    gt = keys > t_s
    g_cnt = jnp.sum(gt.astype(i32), axis=(1, 2), keepdims=True)
    sum_gt = jnp.sum(jnp.where(gt, v, zero))
    t_f = jax.lax.bitcast_convert_type(_signed_key(t_s), f32)
    quota = (num_neg - g_cnt).astype(f32)
    tie = jnp.where(num_neg > g_cnt, quota * t_f, zero)
    loss_c = ce_pos + sum_gt + jnp.sum(tie)

    out_l[...] = out_l[...] + loss_l.reshape(1, 1)
    out_r[...] = out_r[...] + loss_r.reshape(1, 1)
    out_c[...] = out_c[...] + loss_c.reshape(1, 1)
    out_n[...] = out_n[...] + jnp.sum(npos).reshape(1, 1)


@jax.jit
def kernel(loc_data, conf_data, priors, targets):
    pad = _PP - _P
    loc_t = jnp.transpose(loc_data, (0, 2, 1))
    loc_t = jnp.pad(loc_t, ((0, 0), (0, 0), (0, pad)))
    loc_t = loc_t.reshape(_B, 4, _S, _L)
    conf_t = jnp.transpose(conf_data, (0, 2, 1))
    conf_t = jnp.pad(conf_t, ((0, 0), (0, 0), (0, pad)))
    conf_t = conf_t.reshape(_B, _C, _S, _L)
    dummy = jnp.tile(jnp.array([[10.0, 10.0, 1.0, 1.0]], jnp.float32),
                     (pad, 1))
    priors_t = jnp.concatenate([priors, dummy], axis=0).T
    priors_t = priors_t.reshape(4, _S, _L)

    out_shapes = [jax.ShapeDtypeStruct((1, 1), jnp.float32)] * 4
    outs = pl.pallas_call(
        _mbl_body,
        grid=(_B // _M,),
        in_specs=[
            pl.BlockSpec((_M, _O, 5), lambda b: (b, 0, 0)),
            pl.BlockSpec((4, _S, _L), lambda b: (0, 0, 0)),
            pl.BlockSpec((_M, 4, _S, _L), lambda b: (b, 0, 0, 0)),
            pl.BlockSpec((_M, _C, _S, _L), lambda b: (b, 0, 0, 0)),
        ],
        out_specs=[pl.BlockSpec((1, 1), lambda b: (0, 0))] * 4,
        out_shape=out_shapes,
    )(targets, priors_t, loc_t, conf_t)
    ll, lr, lc, n = outs
    n = n[0, 0]
    return (ll[0, 0] / n, lr[0, 0] / n, lc[0, 0] / n)


# speedup vs baseline: 1.7607x; 1.7607x over previous
import numpy as np
import jax
import jax.numpy as jnp
from jax.experimental import pallas as pl

_NUM_CLASSES = 21
_THRESHOLD = 0.5
_NEGPOS_RATIO = 3
_V0 = 0.1
_V1 = 0.2
_B, _P, _C, _O = 32, 8732, 21, 16
_M = 8
_SIGN = -(2 ** 31)


def _signed_key(i):
    return i ^ ((i >> 31) & 0x7FFFFFFF)


def _mbl_body(tgt_ref, prior_ref, loc_ref, conf_ref,
              out_l, out_r, out_c, out_n):
    g = pl.program_id(0)

    @pl.when(g == 0)
    def _init():
        z = jnp.zeros((1, 1), jnp.float32)
        out_l[...] = z
        out_r[...] = z
        out_c[...] = z
        out_n[...] = z

    f32 = jnp.float32
    i32 = jnp.int32
    pcx = prior_ref[0:1, :]
    pcy = prior_ref[1:2, :]
    pw = prior_ref[2:3, :]
    ph = prior_ref[3:4, :]
    px1 = pcx - pw * 0.5
    py1 = pcy - ph * 0.5
    px2 = pcx + pw * 0.5
    py2 = pcy + ph * 0.5
    parea = (px2 - px1) * (py2 - py1)

    pi = jax.lax.broadcasted_iota(i32, (1, _P), 1)

    def tcol(t, col):
        return tgt_ref[t, :, col:col + 1]

    v1 = jnp.full((_M, _P), -1.0, f32)
    v2 = v1
    i1 = jnp.zeros((_M, _P), i32)
    i2 = jnp.zeros((_M, _P), i32)
    t_match = jnp.full((_M, _P), -1, i32)
    big = jnp.int32(_P)
    for t in range(_O):
        tx1 = tcol(t, 0)
        ty1 = tcol(t, 1)
        tx2 = tcol(t, 2)
        ty2 = tcol(t, 3)
        tarea = (tx2 - tx1) * (ty2 - ty1)
        ix = jnp.clip(jnp.minimum(tx2, px2) - jnp.maximum(tx1, px1), 0.0, None)
        iy = jnp.clip(jnp.minimum(ty2, py2) - jnp.maximum(ty1, py1), 0.0, None)
        inter = ix * iy
        ov = inter / (tarea + parea - inter)
        upd1 = ov > v1
        upd2 = jnp.logical_and(ov > v2, jnp.logical_not(upd1))
        v2 = jnp.where(upd1, v1, jnp.where(upd2, ov, v2))
        i2 = jnp.where(upd1, i1, jnp.where(upd2, t, i2))
        v1 = jnp.where(upd1, ov, v1)
        i1 = jnp.where(upd1, t, i1)
        m_t = jnp.max(ov, axis=1, keepdims=True)
        bp = jnp.min(jnp.where(ov == m_t, pi, big),
                     axis=1, keepdims=True)
        t_match = jnp.where(pi == bp, t, t_match)

    forced = t_match >= 0
    ovl = jnp.where(forced, 2.0, v1)
    idxf = jnp.where(forced, t_match, i1)
    sb_idx = i2

    def gather_tree(idx, col):
        bit0 = (idx & 1) == 1
        bit1 = (idx & 2) == 2
        bit2 = (idx & 4) == 4
        bit3 = (idx & 8) == 8
        lvl = [jnp.where(bit0, tcol(2 * j + 1, col), tcol(2 * j, col))
               for j in range(8)]
        lvl = [jnp.where(bit1, lvl[2 * j + 1], lvl[2 * j]) for j in range(4)]
        lvl = [jnp.where(bit2, lvl[2 * j + 1], lvl[2 * j]) for j in range(2)]
        return jnp.where(bit3, lvl[1], lvl[0])

    mx1 = gather_tree(idxf, 0)
    my1 = gather_tree(idxf, 1)
    mx2 = gather_tree(idxf, 2)
    my2 = gather_tree(idxf, 3)
    lbl = gather_tree(idxf, 4)
    gx1 = gather_tree(sb_idx, 0)
    gy1 = gather_tree(sb_idx, 1)
    gx2 = gather_tree(sb_idx, 2)
    gy2 = gather_tree(sb_idx, 3)

    pos = ovl >= _THRESHOLD
    posf = pos.astype(f32)
    npos = jnp.sum(posf, axis=1, keepdims=True)

    ecx = ((mx1 + mx2) * 0.5 - pcx) / (_V0 * pw)
    ecy = ((my1 + my2) * 0.5 - pcy) / (_V0 * ph)
    ew = jnp.log((mx2 - mx1) / pw) / _V1
    eh = jnp.log((my2 - my1) / ph) / _V1
    ld0 = loc_ref[0]
    ld1 = loc_ref[1]
    ld2 = loc_ref[2]
    ld3 = loc_ref[3]

    sl1 = jnp.zeros((_M, _P), f32)
    for ld, e in ((ld0, ecx), (ld1, ecy), (ld2, ew), (ld3, eh)):
        d = ld - e
        ad = jnp.abs(d)
        sl1 = sl1 + jnp.where(ad < 1.0, 0.5 * d * d, ad - 0.5)
    loss_l = jnp.sum(sl1 * posf)

    dcx = pcx + (ld0 * _V0) * pw
    dcy = pcy + (ld1 * _V0) * ph
    dw = pw * jnp.exp(ld2 * _V1)
    dh = ph * jnp.exp(ld3 * _V1)
    dx1 = dcx - dw * 0.5
    dx2 = dx1 + dw
    dy1 = dcy - dh * 0.5
    dy2 = dy1 + dh
    iw = jnp.clip(jnp.minimum(gx2, dx2) - jnp.maximum(gx1, dx1), 0.0, None)
    ih = jnp.clip(jnp.minimum(gy2, dy2) - jnp.maximum(gy1, dy1), 0.0, None)
    garea = (gx2 - gx1) * (gy2 - gy1)
    iog = (iw * ih) / garea
    rep = -jnp.log(jnp.maximum(1.0 - iog, 0.0) + jnp.float32(1e-10))
    loss_r = jnp.sum(rep * posf)

    cmax = conf_ref[0]
    for c in range(1, _C):
        cmax = jnp.maximum(cmax, conf_ref[c])
    ssum = jnp.zeros((_M, _P), f32)
    for c in range(_C):
        ssum = ssum + jnp.exp(conf_ref[c] - cmax)
    lse = jnp.log(ssum) + cmax
    conf_t = jnp.where(pos, lbl.astype(i32) + 1, 0)
    cb0 = (conf_t & 1) == 1
    cb1 = (conf_t & 2) == 2
    cb2 = (conf_t & 4) == 4
    cb3 = (conf_t & 8) == 8
    cb4 = (conf_t & 16) == 16
    lvl = [jnp.where(cb0, conf_ref[min(2 * j + 1, _C - 1)],
                     conf_ref[2 * j]) for j in range(11)]
    lvl = [jnp.where(cb1, lvl[2 * j + 1], lvl[2 * j]) for j in range(5)] \
        + [lvl[10]]
    lvl = [jnp.where(cb2, lvl[2 * j + 1], lvl[2 * j]) for j in range(3)]
    lvl = [jnp.where(cb3, lvl[1], lvl[0]), lvl[2]]
    csel = jnp.where(cb4, lvl[1], lvl[0])
    ce = lse - csel
    zero = jnp.zeros((), f32)
    ce_pos = jnp.sum(jnp.where(pos, ce, zero))

    v = jnp.where(pos, zero, ce)
    keys = _signed_key(jax.lax.bitcast_convert_type(v, i32))
    npos_i = npos.astype(i32)
    num_neg = jnp.minimum(_NEGPOS_RATIO * npos_i, _P - 1)

    x_bits = jnp.zeros((_M, 1), i32)
    for bit in range(31, -1, -1):
        m = 1 << bit
        if m >= 2 ** 31:
            m -= 2 ** 32
        trial = x_bits | jnp.int32(m)
        trial_s = trial ^ jnp.int32(_SIGN)
        cnt = jnp.sum((keys >= trial_s).astype(i32),
                      axis=1, keepdims=True)
        x_bits = jnp.where(cnt >= num_neg, trial, x_bits)
    t_s = x_bits ^ jnp.int32(_SIGN)

    gt = keys > t_s
    g_cnt = jnp.sum(gt.astype(i32), axis=1, keepdims=True)
    sum_gt = jnp.sum(jnp.where(gt, v, zero))
    t_f = jax.lax.bitcast_convert_type(_signed_key(t_s), f32)
    quota = (num_neg - g_cnt).astype(f32)
    tie = jnp.where(num_neg > g_cnt, quota * t_f, zero)
    loss_c = ce_pos + sum_gt + jnp.sum(tie)

    out_l[...] = out_l[...] + loss_l.reshape(1, 1)
    out_r[...] = out_r[...] + loss_r.reshape(1, 1)
    out_c[...] = out_c[...] + loss_c.reshape(1, 1)
    out_n[...] = out_n[...] + jnp.sum(npos).reshape(1, 1)


@jax.jit
def kernel(loc_data, conf_data, priors, targets):
    loc_t = jnp.transpose(loc_data, (2, 0, 1))
    conf_t = jnp.transpose(conf_data, (2, 0, 1))
    tgt_t = jnp.transpose(targets, (1, 0, 2))
    priors_t = priors.T

    out_shapes = [jax.ShapeDtypeStruct((1, 1), jnp.float32)] * 4
    outs = pl.pallas_call(
        _mbl_body,
        grid=(_B // _M,),
        in_specs=[
            pl.BlockSpec((_O, _M, 5), lambda b: (0, b, 0)),
            pl.BlockSpec((4, _P), lambda b: (0, 0)),
            pl.BlockSpec((4, _M, _P), lambda b: (0, b, 0)),
            pl.BlockSpec((_C, _M, _P), lambda b: (0, b, 0)),
        ],
        out_specs=[pl.BlockSpec((1, 1), lambda b: (0, 0))] * 4,
        out_shape=out_shapes,
    )(tgt_t, priors_t, loc_t, conf_t)
    ll, lr, lc, n = outs
    n = n[0, 0]
    return (ll[0, 0] / n, lr[0, 0] / n, lc[0, 0] / n)
